# baseline (speedup 1.0000x reference)
import functools

import jax
import jax.numpy as jnp
from jax import lax
from jax.experimental import pallas as pl
from jax.experimental.pallas import tpu as pltpu
from jax.experimental.pallas import tpu_sc as plsc

NUM_EMBEDDINGS = 1024
EMBEDDING_DIM = 64
COMMITMENT_COST = 0.25
KL_WEIGHT = 1.0

ROWS_PER_BLOCK = 1024
N_TOKENS = 16 * 1024
N_TOKEN_CHUNKS = 2
CHUNK_TOKENS = N_TOKENS // N_TOKEN_CHUNKS

SC_CORES = 2
SC_SUBCORES = 16
SC_WORKERS = SC_CORES * SC_SUBCORES
B_PER_W = CHUNK_TOKENS // SC_WORKERS
GATHER_CHUNK = 128
N_CHUNKS = B_PER_W // GATHER_CHUNK


def _argmin_body(x_ref, emb_ref, idx_ref, lsum_ref, col_ref, acc_ref):
    i = pl.program_id(0)
    nblocks = pl.num_programs(0)

    x = x_ref[:, :]
    emb = emb_ref[:, :]

    @pl.when(i == 0)
    def _():
        col_ref[:, :] = lax.broadcasted_iota(
            jnp.int32, (ROWS_PER_BLOCK, NUM_EMBEDDINGS), 1)
        acc_ref[0] = 0.0

    xsq = jnp.sum(x * x, axis=1, keepdims=True)
    esq = jnp.sum(emb * emb, axis=1, keepdims=True)
    mm2 = lax.dot_general(
        x * (-2.0), emb, (((1,), (1,)), ((), ())),
        preferred_element_type=jnp.float32)
    d = (xsq + esq.reshape(1, NUM_EMBEDDINGS)) + mm2

    min_d = jnp.min(d, axis=1, keepdims=True)
    idx = jnp.min(jnp.where(d == min_d, col_ref[:, :], NUM_EMBEDDINGS),
                  axis=1, keepdims=True)
    idx_ref[:, :] = idx

    acc_ref[0] += jnp.sum(min_d)

    @pl.when(i == nblocks - 1)
    def _():
        lsum_ref[0, 0] = acc_ref[0]


def _gather_count_body(table_hbm, idx_hbm, out_hbm, cnt_hbm,
                       idx_v, idx_flat_v, rows_v, ones_v, zcnt_v,
                       shared_cnt, sem):
    cid = lax.axis_index("c")
    sid = lax.axis_index("s")
    wid = sid * SC_CORES + cid
    base = wid * B_PER_W
    for c in range(N_CHUNKS):
        pltpu.sync_copy(idx_hbm.at[pl.ds(base + c * GATHER_CHUNK,
                                         GATHER_CHUNK)], idx_v.at[c])
    pltpu.sync_copy(idx_hbm.at[pl.ds(base, B_PER_W)], idx_flat_v)
    cps = [pltpu.async_copy(table_hbm.at[idx_v.at[c]], rows_v.at[c], sem)
           for c in range(N_CHUNKS)]
    L = 16
    for j in range(NUM_EMBEDDINGS // L):
        zcnt_v[pl.ds(j * L, L)] = jnp.zeros((L,), jnp.float32)
    for j in range(B_PER_W // L):
        ones_v[pl.ds(j * L, L)] = jnp.ones((L,), jnp.float32)

    @pl.when(sid == 0)
    def _():
        pltpu.sync_copy(zcnt_v, shared_cnt)
    plsc.subcore_barrier()
    pltpu.sync_copy(ones_v, shared_cnt.at[idx_flat_v], add=True)
    plsc.subcore_barrier()

    @pl.when(sid == 0)
    def _():
        pltpu.sync_copy(shared_cnt, cnt_hbm.at[cid])
    for c in range(N_CHUNKS):
        cps[c].wait()
        pltpu.sync_copy(rows_v.at[c],
                        out_hbm.at[pl.ds(base + c * GATHER_CHUNK,
                                         GATHER_CHUNK)])


def _kl_body(cnt_ref, prior_ref, lsum_ref, loss_ref):
    counts = jnp.sum(cnt_ref[:, :], axis=0, keepdims=True)
    probs = counts / float(N_TOKENS)
    prior = prior_ref[:, :]
    kl = jnp.sum(probs * (jnp.log(probs + 1e-10) - jnp.log(prior + 1e-10)))
    e_latent = lsum_ref[0, 0] / float(N_TOKENS * EMBEDDING_DIM)
    loss_ref[0, 0] = (1.0 + COMMITMENT_COST) * e_latent + KL_WEIGHT * kl


def _tc_argmin(flat_x, embeddings, ci):
    nblocks = CHUNK_TOKENS // ROWS_PER_BLOCK
    base = ci * nblocks
    return pl.pallas_call(
        _argmin_body,
        grid=(nblocks,),
        in_specs=[
            pl.BlockSpec((ROWS_PER_BLOCK, EMBEDDING_DIM),
                         lambda i: (base + i, 0)),
            pl.BlockSpec((NUM_EMBEDDINGS, EMBEDDING_DIM), lambda i: (0, 0)),
        ],
        out_specs=[
            pl.BlockSpec((ROWS_PER_BLOCK, 1), lambda i: (i, 0)),
            pl.BlockSpec(memory_space=pltpu.SMEM),
        ],
        out_shape=[
            jax.ShapeDtypeStruct((CHUNK_TOKENS, 1), jnp.int32),
            jax.ShapeDtypeStruct((1, 1), jnp.float32),
        ],
        scratch_shapes=[
            pltpu.VMEM((ROWS_PER_BLOCK, NUM_EMBEDDINGS), jnp.int32),
            pltpu.SMEM((1,), jnp.float32),
        ],
    )(flat_x, embeddings)


_sc_gather = functools.partial(
    pl.kernel,
    mesh=plsc.VectorSubcoreMesh(core_axis_name="c", subcore_axis_name="s"),
    compiler_params=pltpu.CompilerParams(use_tc_tiling_on_sc=False),
    out_type=[
        jax.ShapeDtypeStruct((CHUNK_TOKENS, EMBEDDING_DIM), jnp.float32),
        jax.ShapeDtypeStruct((SC_CORES, NUM_EMBEDDINGS), jnp.float32),
    ],
    scratch_types=[
        pltpu.VMEM((N_CHUNKS, GATHER_CHUNK), jnp.int32),
        pltpu.VMEM((B_PER_W,), jnp.int32),
        pltpu.VMEM((N_CHUNKS, GATHER_CHUNK, EMBEDDING_DIM), jnp.float32),
        pltpu.VMEM((B_PER_W,), jnp.float32),
        pltpu.VMEM((NUM_EMBEDDINGS,), jnp.float32),
        pltpu.VMEM_SHARED((NUM_EMBEDDINGS,), jnp.float32),
        pltpu.SemaphoreType.DMA,
    ],
)(_gather_count_body)


def kernel(x, embeddings, running_prior):
    flat_x = x.reshape(-1, EMBEDDING_DIM)
    prior2d = running_prior.reshape(1, NUM_EMBEDDINGS)

    qs, cnts, lsums = [], [], []
    for ci in range(N_TOKEN_CHUNKS):
        idx, lsum = _tc_argmin(flat_x, embeddings, ci)
        q, cnt = _sc_gather(embeddings, idx.reshape(CHUNK_TOKENS))
        qs.append(q)
        cnts.append(cnt)
        lsums.append(lsum)

    cnt_all = jnp.concatenate(cnts, axis=0)
    lsum_all = lsums[0] + lsums[1]

    loss = pl.pallas_call(
        _kl_body,
        in_specs=[
            pl.BlockSpec((N_TOKEN_CHUNKS * SC_CORES, NUM_EMBEDDINGS),
                         lambda: (0, 0)),
            pl.BlockSpec((1, NUM_EMBEDDINGS), lambda: (0, 0)),
            pl.BlockSpec(memory_space=pltpu.SMEM),
        ],
        out_specs=pl.BlockSpec(memory_space=pltpu.SMEM),
        out_shape=jax.ShapeDtypeStruct((1, 1), jnp.float32),
    )(cnt_all, prior2d, lsum_all)

    quantized = jnp.concatenate(qs, axis=0)
    return quantized.reshape(x.shape), loss.reshape(())


# speedup vs baseline: 1.5830x; 1.5830x over previous
import jax
import jax.numpy as jnp
from jax.experimental import pallas as pl
from jax.experimental.pallas import tpu as pltpu

NUM_EMBEDDINGS = 1024
EMBEDDING_DIM = 64
COMMITMENT_COST = 0.25
KL_WEIGHT = 1.0

ROWS_PER_BLOCK = 2048
N_TOKENS = 16 * 1024


def _vq_body(x_ref, emb_ref, prior_ref, q_ref, loss_ref,
             col_ref, counts_ref, losssum_ref):
    i = pl.program_id(0)
    nblocks = pl.num_programs(0)

    x = x_ref[:, :]
    emb = emb_ref[:, :]

    @pl.when(i == 0)
    def _():
        col_ref[:, :] = jax.lax.broadcasted_iota(
            jnp.int32, (ROWS_PER_BLOCK, NUM_EMBEDDINGS), 1)
        counts_ref[:, :] = jnp.zeros_like(counts_ref)
        losssum_ref[0] = 0.0

    xsq = jnp.sum(x * x, axis=1, keepdims=True)
    esq = jnp.sum(emb * emb, axis=1, keepdims=True)
    mm2 = jax.lax.dot_general(
        x * (-2.0), emb, (((1,), (1,)), ((), ())),
        preferred_element_type=jnp.float32)
    d = (xsq + esq.reshape(1, NUM_EMBEDDINGS)) + mm2

    col = col_ref[:, :]
    min_d = jnp.min(d, axis=1, keepdims=True)
    idx = jnp.min(jnp.where(d == min_d, col, NUM_EMBEDDINGS),
                  axis=1, keepdims=True)

    onehot = (col == idx).astype(jnp.float32)
    q = jax.lax.dot_general(
        onehot, emb, (((1,), (0,)), ((), ())),
        preferred_element_type=jnp.float32)
    q_ref[:, :] = q

    ones_row = jnp.ones((8, ROWS_PER_BLOCK), jnp.float32)
    counts_ref[:, :] += jax.lax.dot_general(
        ones_row, onehot, (((1,), (0,)), ((), ())),
        preferred_element_type=jnp.float32)[0:1, :]
    losssum_ref[0] += jnp.sum(min_d)

    @pl.when(i == nblocks - 1)
    def _():
        probs = counts_ref[:, :] / float(N_TOKENS)
        prior = prior_ref[:, :]
        kl = jnp.sum(probs * (jnp.log(probs + 1e-10) - jnp.log(prior + 1e-10)))
        e_latent = losssum_ref[0] / float(N_TOKENS * EMBEDDING_DIM)
        loss_ref[0, 0] = (1.0 + COMMITMENT_COST) * e_latent + KL_WEIGHT * kl


def kernel(x, embeddings, running_prior):
    flat_x = x.reshape(-1, EMBEDDING_DIM)
    prior2d = running_prior.reshape(1, NUM_EMBEDDINGS)
    nblocks = N_TOKENS // ROWS_PER_BLOCK

    quantized, loss = pl.pallas_call(
        _vq_body,
        grid=(nblocks,),
        in_specs=[
            pl.BlockSpec((ROWS_PER_BLOCK, EMBEDDING_DIM), lambda i: (i, 0)),
            pl.BlockSpec((NUM_EMBEDDINGS, EMBEDDING_DIM), lambda i: (0, 0)),
            pl.BlockSpec((1, NUM_EMBEDDINGS), lambda i: (0, 0)),
        ],
        out_specs=[
            pl.BlockSpec((ROWS_PER_BLOCK, EMBEDDING_DIM), lambda i: (i, 0)),
            pl.BlockSpec(memory_space=pltpu.SMEM),
        ],
        out_shape=[
            jax.ShapeDtypeStruct((N_TOKENS, EMBEDDING_DIM), jnp.float32),
            jax.ShapeDtypeStruct((1, 1), jnp.float32),
        ],
        scratch_shapes=[
            pltpu.VMEM((ROWS_PER_BLOCK, NUM_EMBEDDINGS), jnp.int32),
            pltpu.VMEM((1, NUM_EMBEDDINGS), jnp.float32),
            pltpu.SMEM((1,), jnp.float32),
        ],
    )(flat_x, embeddings, prior2d)

    return quantized.reshape(x.shape), loss.reshape(())


# speedup vs baseline: 1.5907x; 1.0049x over previous
import jax
import jax.numpy as jnp
from jax.experimental import pallas as pl
from jax.experimental.pallas import tpu as pltpu

NUM_EMBEDDINGS = 1024
EMBEDDING_DIM = 64
COMMITMENT_COST = 0.25
KL_WEIGHT = 1.0

ROWS_PER_BLOCK = 4096
N_TOKENS = 16 * 1024


def _vq_body(x_ref, emb_ref, prior_ref, q_ref, loss_ref,
             col_ref, counts_ref, losssum_ref):
    i = pl.program_id(0)
    nblocks = pl.num_programs(0)

    x = x_ref[:, :]
    emb = emb_ref[:, :]

    @pl.when(i == 0)
    def _():
        col_ref[:, :] = jax.lax.broadcasted_iota(
            jnp.int32, (ROWS_PER_BLOCK, NUM_EMBEDDINGS), 1)
        counts_ref[:, :] = jnp.zeros_like(counts_ref)
        losssum_ref[0] = 0.0

    xsq = jnp.sum(x * x, axis=1, keepdims=True)
    esq = jnp.sum(emb * emb, axis=1, keepdims=True)
    mm2 = jax.lax.dot_general(
        x * (-2.0), emb, (((1,), (1,)), ((), ())),
        preferred_element_type=jnp.float32)
    d = (xsq + esq.reshape(1, NUM_EMBEDDINGS)) + mm2

    col = col_ref[:, :]
    min_d = jnp.min(d, axis=1, keepdims=True)
    idx = jnp.min(jnp.where(d == min_d, col, NUM_EMBEDDINGS),
                  axis=1, keepdims=True)

    onehot = (col == idx).astype(jnp.float32)
    q = jax.lax.dot_general(
        onehot, emb, (((1,), (0,)), ((), ())),
        preferred_element_type=jnp.float32)
    q_ref[:, :] = q

    ones_row = jnp.ones((8, ROWS_PER_BLOCK), jnp.float32)
    counts_ref[:, :] += jax.lax.dot_general(
        ones_row, onehot, (((1,), (0,)), ((), ())),
        preferred_element_type=jnp.float32)[0:1, :]
    losssum_ref[0] += jnp.sum(min_d)

    @pl.when(i == nblocks - 1)
    def _():
        probs = counts_ref[:, :] / float(N_TOKENS)
        prior = prior_ref[:, :]
        kl = jnp.sum(probs * (jnp.log(probs + 1e-10) - jnp.log(prior + 1e-10)))
        e_latent = losssum_ref[0] / float(N_TOKENS * EMBEDDING_DIM)
        loss_ref[0, 0] = (1.0 + COMMITMENT_COST) * e_latent + KL_WEIGHT * kl


def kernel(x, embeddings, running_prior):
    flat_x = x.reshape(-1, EMBEDDING_DIM)
    prior2d = running_prior.reshape(1, NUM_EMBEDDINGS)
    nblocks = N_TOKENS // ROWS_PER_BLOCK

    quantized, loss = pl.pallas_call(
        _vq_body,
        grid=(nblocks,),
        in_specs=[
            pl.BlockSpec((ROWS_PER_BLOCK, EMBEDDING_DIM), lambda i: (i, 0)),
            pl.BlockSpec((NUM_EMBEDDINGS, EMBEDDING_DIM), lambda i: (0, 0)),
            pl.BlockSpec((1, NUM_EMBEDDINGS), lambda i: (0, 0)),
        ],
        out_specs=[
            pl.BlockSpec((ROWS_PER_BLOCK, EMBEDDING_DIM), lambda i: (i, 0)),
            pl.BlockSpec(memory_space=pltpu.SMEM),
        ],
        out_shape=[
            jax.ShapeDtypeStruct((N_TOKENS, EMBEDDING_DIM), jnp.float32),
            jax.ShapeDtypeStruct((1, 1), jnp.float32),
        ],
        scratch_shapes=[
            pltpu.VMEM((ROWS_PER_BLOCK, NUM_EMBEDDINGS), jnp.int32),
            pltpu.VMEM((1, NUM_EMBEDDINGS), jnp.float32),
            pltpu.SMEM((1,), jnp.float32),
        ],
    )(flat_x, embeddings, prior2d)

    return quantized.reshape(x.shape), loss.reshape(())
